# baseline (speedup 1.0000x reference)
import functools

import jax
import jax.numpy as jnp
from jax import lax
from jax.experimental import pallas as pl
from jax.experimental.pallas import tpu as pltpu
from jax.experimental.pallas import tpu_sc as plsc

NC = 2
NS = 16
NW = NC * NS
LANES = 16
CB = 32


def _mm_body(u_ref, wt_ref, b_ref, out_ref):
    out_ref[...] = (
        jnp.dot(u_ref[...], wt_ref[...], preferred_element_type=jnp.float32)
        + b_ref[...]
    )


def _matmul(ub, wt, bias):
    B, BERT = ub.shape
    D = wt.shape[1]
    BLK = 2048
    grid = (B // BLK,)
    return pl.pallas_call(
        _mm_body,
        grid=grid,
        in_specs=[
            pl.BlockSpec((BLK, BERT), lambda i: (i, 0)),
            pl.BlockSpec((BERT, D), lambda i: (0, 0)),
            pl.BlockSpec((1, D), lambda i: (0, 0)),
        ],
        out_specs=pl.BlockSpec((BLK, D), lambda i: (i, 0)),
        out_shape=jax.ShapeDtypeStruct((B, D), jnp.float32),
    )(ub, wt, bias)


def _make_sc_kernel(B, K, D, V):
    per_w = B // NW
    nchunk = per_w // CB
    nir = (CB * K) // 128

    mesh = plsc.VectorSubcoreMesh(core_axis_name="c", subcore_axis_name="s")

    @functools.partial(
        pl.kernel,
        mesh=mesh,
        compiler_params=pltpu.CompilerParams(use_tc_tiling_on_sc=False),
        out_type=jax.ShapeDtypeStruct((B * (K + 1), LANES), jnp.float32),
        scratch_types=[
            pltpu.VMEM((CB,), jnp.int32),
            pltpu.VMEM((CB * K,), jnp.int32),
            pltpu.VMEM((CB, D), jnp.float32),
            pltpu.VMEM((CB, D), jnp.float32),
            pltpu.VMEM((CB * K, D), jnp.float32),
            pltpu.VMEM((CB, LANES), jnp.float32),
            pltpu.VMEM((CB * K, LANES), jnp.float32),
            pltpu.SemaphoreType.DMA,
        ],
    )
    def sc_kernel(pos_hbm, neg_hbm, emb_hbm, table_hbm, out_hbm,
                  posidx_v, negidx_v, emb_v, rpos_v, rneg_v,
                  ppos_v, pneg_v, sem):
        wid = lax.axis_index("s") * NC + lax.axis_index("c")

        def chunk_body(n, _):
            cb_base = wid * per_w + n * CB

            h0 = pltpu.async_copy(
                pos_hbm.at[pl.ds(cb_base, CB)], posidx_v, sem)
            h1 = pltpu.async_copy(
                neg_hbm.at[pl.ds(cb_base * K, CB * K)], negidx_v, sem)
            h2 = pltpu.async_copy(
                emb_hbm.at[pl.ds(cb_base, CB)], emb_v, sem)
            h0.wait()
            h1.wait()
            g0 = pltpu.async_copy(table_hbm.at[posidx_v], rpos_v, sem)
            gs = [
                pltpu.async_copy(
                    table_hbm.at[negidx_v.at[pl.ds(j * 128, 128)]],
                    rneg_v.at[pl.ds(j * 128, 128)], sem)
                for j in range(nir)
            ]
            h2.wait()
            g0.wait()
            for g in gs:
                g.wait()

            def b_body(b, carry):
                e = [emb_v[b, pl.ds(g * LANES, LANES)] for g in range(4)]
                r = [rpos_v[b, pl.ds(g * LANES, LANES)] for g in range(4)]
                ppos_v[b, :] = ((e[0] * r[0] + e[1] * r[1])
                                + (e[2] * r[2] + e[3] * r[3]))
                for k in range(K):
                    row = b * K + k
                    rn = [rneg_v[row, pl.ds(g * LANES, LANES)]
                          for g in range(4)]
                    pneg_v[row, :] = ((e[0] * rn[0] + e[1] * rn[1])
                                      + (e[2] * rn[2] + e[3] * rn[3]))
                return carry

            lax.fori_loop(0, CB, b_body, 0)

            pltpu.sync_copy(ppos_v, out_hbm.at[pl.ds(cb_base, CB)])
            pltpu.sync_copy(
                pneg_v, out_hbm.at[pl.ds(B + cb_base * K, CB * K)])
            return 0

        lax.fori_loop(0, nchunk, chunk_body, 0)

    return sc_kernel


def _red_body(p_ref, out_ref):
    step = pl.program_id(0)
    ii = lax.broadcasted_iota(jnp.int32, (128, 8), 0)
    gg = lax.broadcasted_iota(jnp.int32, (128, 8), 1)
    sel = ((ii // LANES) == gg).astype(jnp.float32)
    s = jnp.dot(p_ref[...], sel, preferred_element_type=jnp.float32)
    sign = jnp.where(step == 0, 1.0, -1.0).astype(jnp.float32)
    ls = jax.nn.log_sigmoid(sign * s)

    @pl.when(step == 0)
    def _():
        out_ref[0] = 0.0

    out_ref[0] = out_ref[0] - jnp.sum(ls)


def _reduce(p2d, nsteps):
    rows = p2d.shape[0] // nsteps
    return pl.pallas_call(
        _red_body,
        grid=(nsteps,),
        in_specs=[pl.BlockSpec((rows, 128), lambda i: (i, 0))],
        out_specs=pl.BlockSpec(memory_space=pltpu.SMEM),
        out_shape=jax.ShapeDtypeStruct((1,), jnp.float32),
    )(p2d)


def kernel(pos_u, pos_v, neg_v, u_bert, v_table, W_w, W_b):
    B, K = neg_v.shape
    V, D = v_table.shape
    BERT = u_bert.shape[-1]

    ub = u_bert.reshape(B, BERT)
    emb = _matmul(ub, W_w.T, W_b.reshape(1, D))

    pos_i = pos_v.astype(jnp.int32)
    neg_i = neg_v.astype(jnp.int32).reshape(B * K)

    sc = _make_sc_kernel(B, K, D, V)
    psum = sc(pos_i, neg_i, emb, v_table)

    p2d = psum.reshape((B * (K + 1) * LANES) // 128, 128)
    total = _reduce(p2d, K + 1)
    return total[0]


# speedup vs baseline: 1.0645x; 1.0645x over previous
import functools

import jax
import jax.numpy as jnp
from jax import lax
from jax.experimental import pallas as pl
from jax.experimental.pallas import tpu as pltpu
from jax.experimental.pallas import tpu_sc as plsc

NC = 2
NS = 16
NW = NC * NS
LANES = 16
CB = 64


def _mm_body(u_ref, wt_ref, b_ref, out_ref):
    u = u_ref[...].reshape(u_ref.shape[0], u_ref.shape[2])
    out_ref[...] = (
        jnp.dot(u.astype(jnp.bfloat16), wt_ref[...],
                preferred_element_type=jnp.float32)
        + b_ref[...]
    )


def _matmul(u3, wt, bias):
    B, _, BERT = u3.shape
    D = wt.shape[1]
    BLK = 2048
    grid = (B // BLK,)
    return pl.pallas_call(
        _mm_body,
        grid=grid,
        in_specs=[
            pl.BlockSpec((BLK, 1, BERT), lambda i: (i, 0, 0)),
            pl.BlockSpec((BERT, D), lambda i: (0, 0)),
            pl.BlockSpec((1, D), lambda i: (0, 0)),
        ],
        out_specs=pl.BlockSpec((BLK, D), lambda i: (i, 0)),
        out_shape=jax.ShapeDtypeStruct((B, D), jnp.float32),
    )(u3, wt, bias)


def _make_sc_kernel(B, K, D, V):
    per_w = B // NW
    nchunk = per_w // CB
    nir = (CB * K) // 128

    mesh = plsc.VectorSubcoreMesh(core_axis_name="c", subcore_axis_name="s")

    @functools.partial(
        pl.kernel,
        mesh=mesh,
        compiler_params=pltpu.CompilerParams(use_tc_tiling_on_sc=False),
        out_type=jax.ShapeDtypeStruct((B * (K + 1), LANES), jnp.float32),
        scratch_types=[
            pltpu.VMEM((CB,), jnp.int32),
            pltpu.VMEM((CB * K,), jnp.int32),
            pltpu.VMEM((CB, D), jnp.float32),
            pltpu.VMEM((CB, D), jnp.float32),
            pltpu.VMEM((CB * K, D), jnp.float32),
            pltpu.VMEM((CB, LANES), jnp.float32),
            pltpu.VMEM((CB * K, LANES), jnp.float32),
            pltpu.SemaphoreType.DMA,
        ],
    )
    def sc_kernel(pos_hbm, neg_hbm, emb_hbm, table_hbm, out_hbm,
                  posidx_v, negidx_v, emb_v, rpos_v, rneg_v,
                  ppos_v, pneg_v, sem):
        wid = lax.axis_index("s") * NC + lax.axis_index("c")

        def chunk_body(n, _):
            cb_base = wid * per_w + n * CB

            h0 = pltpu.async_copy(
                pos_hbm.at[pl.ds(cb_base, CB)], posidx_v, sem)
            h1 = pltpu.async_copy(
                neg_hbm.at[pl.ds(cb_base * K, CB * K)], negidx_v, sem)
            h2 = pltpu.async_copy(
                emb_hbm.at[pl.ds(cb_base, CB)], emb_v, sem)
            h0.wait()
            h1.wait()
            g0 = pltpu.async_copy(table_hbm.at[posidx_v], rpos_v, sem)
            gs = [
                pltpu.async_copy(
                    table_hbm.at[negidx_v.at[pl.ds(j * 128, 128)]],
                    rneg_v.at[pl.ds(j * 128, 128)], sem)
                for j in range(nir)
            ]
            h2.wait()
            g0.wait()
            for g in gs:
                g.wait()

            def b_body(b, carry):
                e = [emb_v[b, pl.ds(g * LANES, LANES)] for g in range(4)]
                r = [rpos_v[b, pl.ds(g * LANES, LANES)] for g in range(4)]
                ppos_v[b, :] = ((e[0] * r[0] + e[1] * r[1])
                                + (e[2] * r[2] + e[3] * r[3]))
                for k in range(K):
                    row = b * K + k
                    rn = [rneg_v[row, pl.ds(g * LANES, LANES)]
                          for g in range(4)]
                    pneg_v[row, :] = ((e[0] * rn[0] + e[1] * rn[1])
                                      + (e[2] * rn[2] + e[3] * rn[3]))
                return carry

            lax.fori_loop(0, CB, b_body, 0)

            pltpu.sync_copy(ppos_v, out_hbm.at[pl.ds(cb_base, CB)])
            pltpu.sync_copy(
                pneg_v, out_hbm.at[pl.ds(B + cb_base * K, CB * K)])
            return 0

        lax.fori_loop(0, nchunk, chunk_body, 0)

    return sc_kernel


def _red_body(p_ref, out_ref):
    step = pl.program_id(0)
    ii = lax.broadcasted_iota(jnp.int32, (128, 8), 0)
    gg = lax.broadcasted_iota(jnp.int32, (128, 8), 1)
    sel = ((ii // LANES) == gg).astype(jnp.float32)
    s = jnp.dot(p_ref[...], sel, preferred_element_type=jnp.float32)
    sign = jnp.where(step == 0, 1.0, -1.0).astype(jnp.float32)
    ls = jax.nn.log_sigmoid(sign * s)

    @pl.when(step == 0)
    def _():
        out_ref[0] = 0.0

    out_ref[0] = out_ref[0] - jnp.sum(ls)


def _reduce(p2d, nsteps):
    rows = p2d.shape[0] // nsteps
    return pl.pallas_call(
        _red_body,
        grid=(nsteps,),
        in_specs=[pl.BlockSpec((rows, 128), lambda i: (i, 0))],
        out_specs=pl.BlockSpec(memory_space=pltpu.SMEM),
        out_shape=jax.ShapeDtypeStruct((1,), jnp.float32),
    )(p2d)


def kernel(pos_u, pos_v, neg_v, u_bert, v_table, W_w, W_b):
    B, K = neg_v.shape
    V, D = v_table.shape
    BERT = u_bert.shape[-1]

    emb = _matmul(u_bert, W_w.T, W_b.reshape(1, D))

    pos_i = pos_v.astype(jnp.int32)
    neg_i = neg_v.astype(jnp.int32).reshape(B * K)

    sc = _make_sc_kernel(B, K, D, V)
    psum = sc(pos_i, neg_i, emb, v_table)

    p2d = psum.reshape((B * (K + 1) * LANES) // 128, 128)
    total = _reduce(p2d, K + 1)
    return total[0]
